# baseline (speedup 1.0000x reference)
import functools

import jax
import jax.numpy as jnp
from jax import lax
from jax.experimental import pallas as pl
from jax.experimental.pallas import tpu as pltpu
from jax.experimental.pallas import tpu_sc as plsc

BATCH = 16384
DIM = 128

_INFO = plsc.get_sparse_core_info()
_NC = _INFO.num_cores
_NS = _INFO.num_subcores
_NW = _NC * _NS
_BPW = BATCH // _NW
_CHUNK = 128
_NCHUNK = _BPW // _CHUNK


def _sc_gather_body(bu_hbm, bi_hbm, uid_hbm, iid_hbm, out_hbm,
                    uidx_v, iidx_v, bu_v, bi_v, sem_u, sem_i):
    wid = lax.axis_index("s") * _NC + lax.axis_index("c")
    base = wid * _BPW
    pltpu.sync_copy(uid_hbm.at[pl.ds(base, _BPW)], uidx_v)
    pltpu.sync_copy(iid_hbm.at[pl.ds(base, _BPW)], iidx_v)
    copies = []
    for j in range(_NCHUNK):
        sl = pl.ds(j * _CHUNK, _CHUNK)
        copies.append(pltpu.async_copy(bu_hbm.at[uidx_v.at[sl]], bu_v.at[sl], sem_u))
        copies.append(pltpu.async_copy(bi_hbm.at[iidx_v.at[sl]], bi_v.at[sl], sem_i))
    for c in copies:
        c.wait()
    for j in range(_BPW // 16):
        sl = pl.ds(j * 16, 16)
        bu_v[sl] = bu_v[sl] + bi_v[sl]
    pltpu.sync_copy(bu_v, out_hbm.at[pl.ds(base, _BPW)])


def _sc_gather(bu_flat, bi_flat, uid, iid):
    mesh = plsc.VectorSubcoreMesh(core_axis_name="c", subcore_axis_name="s")
    fn = functools.partial(
        pl.kernel,
        mesh=mesh,
        out_type=jax.ShapeDtypeStruct((BATCH,), jnp.float32),
        scratch_types=[
            pltpu.VMEM((_BPW,), jnp.int32),
            pltpu.VMEM((_BPW,), jnp.int32),
            pltpu.VMEM((_BPW,), jnp.float32),
            pltpu.VMEM((_BPW,), jnp.float32),
            pltpu.SemaphoreType.DMA,
            pltpu.SemaphoreType.DMA,
        ],
    )(_sc_gather_body)
    return fn(bu_flat, bi_flat, uid, iid)


def _tc_body(fc_w_ref, fc_b_ref, f_ref, g_ref, o_ref):
    w = fc_w_ref[:, :]
    acc = jnp.sum(f_ref[:, :] * w, axis=1, keepdims=True)
    o_ref[:, :] = acc + g_ref[:, :] + fc_b_ref[0]


_BLK = 2048


def kernel(feature, user_id, item_id, fc_w, fc_b, b_users, b_items):
    uid = user_id.astype(jnp.int32)
    iid = item_id.astype(jnp.int32)
    bu_flat = b_users.reshape(-1)
    bi_flat = b_items.reshape(-1)

    g = _sc_gather(bu_flat, bi_flat, uid, iid)
    g2 = g.reshape(BATCH, 1)

    out = pl.pallas_call(
        _tc_body,
        grid=(BATCH // _BLK,),
        in_specs=[
            pl.BlockSpec((1, DIM), lambda i: (0, 0)),
            pl.BlockSpec(memory_space=pltpu.SMEM),
            pl.BlockSpec((_BLK, DIM), lambda i: (i, 0)),
            pl.BlockSpec((_BLK, 1), lambda i: (i, 0)),
        ],
        out_specs=pl.BlockSpec((_BLK, 1), lambda i: (i, 0)),
        out_shape=jax.ShapeDtypeStruct((BATCH, 1), jnp.float32),
    )(fc_w, fc_b, feature, g2)
    return out
